# Initial kernel scaffold; baseline (speedup 1.0000x reference)
import jax
import jax.numpy as jnp
from jax.experimental import pallas as pl


def kernel(x, edge_index, edge_attr, W, W_ih, W_hh, b_ih, b_hh, mlp_w0, mlp_b0, mlp_w1, mlp_b1, mlp_w2, mlp_b2, out_w, out_b):
    raise NotImplementedError("write your pallas kernel here")


# speedup vs baseline: 3.3208x; 3.3208x over previous
import functools

import jax
import jax.numpy as jnp
from jax import lax
from jax.experimental import pallas as pl
from jax.experimental.pallas import tpu as pltpu
from jax.experimental.pallas import tpu_sc as plsc

_N = 10000
_E = 320000
_D = 128
_NUM_LAYERS = 3

_NC = 2
_NS = 16
_NW = _NC * _NS
_K = 128
_CHUNKS = -(-_E // (_NW * _K))
_EPAD = _NW * _K * _CHUNKS
_EW = _EPAD // _NW
_NPAD = 10240
_RPT = _NPAD // _NS
_ZC = 128
_NZ = _RPT // _ZC

_R = 1000
_G = _N // _R


def _spmm_body(m_hbm, src_hbm, dst_hbm, ea_hbm, out_hbm,
               src_v, dst_v, ea_v, rows_v, bounce_v, acc_sh, sem):
    c = lax.axis_index("c")
    s = lax.axis_index("s")
    wid = c * _NS + s

    def zrow(n, _):
        for j in range(_D // 16):
            bounce_v[n, pl.ds(j * 16, 16)] = jnp.zeros((16,), jnp.float32)
        return 0
    lax.fori_loop(0, _ZC, zrow, 0)
    row0 = s * _RPT
    for z in range(_NZ):
        pltpu.sync_copy(bounce_v, acc_sh.at[pl.ds(row0 + z * _ZC, _ZC)])
    plsc.subcore_barrier()

    ebase = wid * _EW

    def chunk(ci, _):
        off = pl.multiple_of(ebase + ci * _K, 8)
        pltpu.sync_copy(src_hbm.at[pl.ds(off, _K)], src_v)
        pltpu.sync_copy(dst_hbm.at[pl.ds(off, _K)], dst_v)
        pltpu.sync_copy(ea_hbm.at[pl.ds(off, _K)], ea_v)
        pltpu.async_copy(m_hbm.at[src_v], rows_v, sem).wait()

        def grp(g, _):
            e0 = g * 16
            eav = ea_v[pl.ds(e0, 16)]
            for i in range(16):
                w = eav[i]
                for j in range(_D // 16):
                    sl = pl.ds(j * 16, 16)
                    rows_v[e0 + i, sl] = rows_v[e0 + i, sl] * w
            return 0
        lax.fori_loop(0, _K // 16, grp, 0)
        pltpu.sync_copy(rows_v, acc_sh.at[dst_v], add=True)
        return 0
    lax.fori_loop(0, _CHUNKS, chunk, 0)
    plsc.subcore_barrier()

    for z in range(_NZ):
        r0 = row0 + z * _ZC
        pltpu.sync_copy(acc_sh.at[pl.ds(r0, _ZC)], bounce_v)
        pltpu.sync_copy(bounce_v, out_hbm.at[c, pl.ds(r0, _ZC)])


@jax.jit
def _spmm(m, src, dst, ea):
    mesh = plsc.VectorSubcoreMesh(core_axis_name="c", subcore_axis_name="s")
    return pl.kernel(
        _spmm_body,
        out_type=jax.ShapeDtypeStruct((_NC, _NPAD, _D), jnp.float32),
        mesh=mesh,
        scratch_types=[
            pltpu.VMEM((_K,), jnp.int32),
            pltpu.VMEM((_K,), jnp.int32),
            pltpu.VMEM((_K,), jnp.float32),
            pltpu.VMEM((_K, _D), jnp.float32),
            pltpu.VMEM((_ZC, _D), jnp.float32),
            pltpu.VMEM_SHARED((_NPAD, _D), jnp.float32),
            pltpu.SemaphoreType.DMA,
        ],
    )(m, src, dst, ea)


def _matmul_body(x_ref, w_ref, o_ref):
    o_ref[...] = jnp.dot(x_ref[...], w_ref[...],
                         preferred_element_type=jnp.float32)


@jax.jit
def _matmul(x, w):
    return pl.pallas_call(
        _matmul_body,
        grid=(_G,),
        in_specs=[
            pl.BlockSpec((_R, _D), lambda i: (i, 0)),
            pl.BlockSpec((_D, _D), lambda i: (0, 0)),
        ],
        out_specs=pl.BlockSpec((_R, _D), lambda i: (i, 0)),
        out_shape=jax.ShapeDtypeStruct((_N, _D), jnp.float32),
    )(x, w)


def _gru_body(s_ref, h_ref, wih_ref, whh_ref, bih_ref, bhh_ref, wn_ref,
              h_out, m_out):
    agg = s_ref[0] + s_ref[1]
    h = h_ref[...]
    gi = jnp.dot(agg, wih_ref[...], preferred_element_type=jnp.float32)
    gi = gi + bih_ref[...]
    gh = jnp.dot(h, whh_ref[...], preferred_element_type=jnp.float32)
    gh = gh + bhh_ref[...]
    r = jax.nn.sigmoid(gi[:, :_D] + gh[:, :_D])
    z = jax.nn.sigmoid(gi[:, _D:2 * _D] + gh[:, _D:2 * _D])
    n = jnp.tanh(gi[:, 2 * _D:] + r * gh[:, 2 * _D:])
    hn = (1.0 - z) * n + z * h
    h_out[...] = hn
    m_out[...] = jnp.dot(hn, wn_ref[...], preferred_element_type=jnp.float32)


@jax.jit
def _gru(s, h, wih_t, whh_t, bih, bhh, w_next):
    return pl.pallas_call(
        _gru_body,
        grid=(_G,),
        in_specs=[
            pl.BlockSpec((_NC, _R, _D), lambda i: (0, i, 0)),
            pl.BlockSpec((_R, _D), lambda i: (i, 0)),
            pl.BlockSpec((_D, 3 * _D), lambda i: (0, 0)),
            pl.BlockSpec((_D, 3 * _D), lambda i: (0, 0)),
            pl.BlockSpec((1, 3 * _D), lambda i: (0, 0)),
            pl.BlockSpec((1, 3 * _D), lambda i: (0, 0)),
            pl.BlockSpec((_D, _D), lambda i: (0, 0)),
        ],
        out_specs=[
            pl.BlockSpec((_R, _D), lambda i: (i, 0)),
            pl.BlockSpec((_R, _D), lambda i: (i, 0)),
        ],
        out_shape=[
            jax.ShapeDtypeStruct((_N, _D), jnp.float32),
            jax.ShapeDtypeStruct((_N, _D), jnp.float32),
        ],
    )(s, h, wih_t, whh_t, bih, bhh, w_next)


def _mlp_body(h_ref, w0_ref, b0_ref, w1_ref, b1_ref, w2_ref, b2_ref,
              wo_ref, bo_ref, emb_out, lsm_out):
    y = jnp.tanh(jnp.dot(h_ref[...], w0_ref[...],
                         preferred_element_type=jnp.float32) + b0_ref[...])
    y = jnp.tanh(jnp.dot(y, w1_ref[...],
                         preferred_element_type=jnp.float32) + b1_ref[...])
    y = jnp.tanh(jnp.dot(y, w2_ref[...],
                         preferred_element_type=jnp.float32) + b2_ref[...])
    logits = jnp.dot(y, wo_ref[...],
                     preferred_element_type=jnp.float32) + bo_ref[...]
    m = jnp.max(logits, axis=1, keepdims=True)
    lse = jnp.log(jnp.sum(jnp.exp(logits - m), axis=1, keepdims=True)) + m
    emb_out[...] = logits
    lsm_out[...] = logits - lse


@jax.jit
def _mlp(h, w0, b0, w1, b1, w2, b2, wo, bo):
    wspec = pl.BlockSpec((_D, _D), lambda i: (0, 0))
    bspec = pl.BlockSpec((1, _D), lambda i: (0, 0))
    ospec = pl.BlockSpec((_R, _D), lambda i: (i, 0))
    return pl.pallas_call(
        _mlp_body,
        grid=(_G,),
        in_specs=[ospec, wspec, bspec, wspec, bspec, wspec, bspec,
                  wspec, bspec],
        out_specs=[ospec, ospec],
        out_shape=[
            jax.ShapeDtypeStruct((_N, _D), jnp.float32),
            jax.ShapeDtypeStruct((_N, _D), jnp.float32),
        ],
    )(h, w0, b0, w1, b1, w2, b2, wo, bo)


def kernel(x, edge_index, edge_attr, W, W_ih, W_hh, b_ih, b_hh,
           mlp_w0, mlp_b0, mlp_w1, mlp_b1, mlp_w2, mlp_b2, out_w, out_b):
    f32 = jnp.float32
    src = edge_index[0].astype(jnp.int32)
    dst = edge_index[1].astype(jnp.int32)
    ea = edge_attr.astype(f32)
    pad = _EPAD - _E
    src_p = jnp.pad(src, (0, pad))
    dst_p = jnp.pad(dst, (0, pad))
    ea_p = jnp.pad(ea, (0, pad))

    wih_t = W_ih.T.astype(f32)
    whh_t = W_hh.T.astype(f32)
    bih = b_ih.reshape(1, -1).astype(f32)
    bhh = b_hh.reshape(1, -1).astype(f32)

    h = x.astype(f32)
    m = _matmul(h, W[0].astype(f32))
    for i in range(_NUM_LAYERS):
        s = _spmm(m, src_p, dst_p, ea_p)[:, :_N]
        w_next = W[(i + 1) % _NUM_LAYERS].astype(f32)
        h, m = _gru(s, h, wih_t, whh_t, bih, bhh, w_next)

    mh = mlp_w0.shape[0]
    nc = out_w.shape[0]
    w0 = jnp.zeros((_D, _D), f32).at[:, :mh].set(mlp_w0.T)
    b0 = jnp.zeros((1, _D), f32).at[0, :mh].set(mlp_b0)
    w1 = jnp.zeros((_D, _D), f32).at[:mh, :mh].set(mlp_w1.T)
    b1 = jnp.zeros((1, _D), f32).at[0, :mh].set(mlp_b1)
    w2 = jnp.zeros((_D, _D), f32).at[:mh, :mh].set(mlp_w2.T)
    b2 = jnp.zeros((1, _D), f32).at[0, :mh].set(mlp_b2)
    wo = jnp.zeros((_D, _D), f32).at[:mh, :nc].set(out_w.T)
    bo = jnp.full((1, _D), -1e9, f32).at[0, :nc].set(out_b)
    emb, lsm = _mlp(h, w0, b0, w1, b1, w2, b2, wo, bo)
    return emb[:, :nc], lsm[:, :nc]


# speedup vs baseline: 3.4601x; 1.0419x over previous
import functools

import jax
import jax.numpy as jnp
from jax import lax
from jax.experimental import pallas as pl
from jax.experimental.pallas import tpu as pltpu
from jax.experimental.pallas import tpu_sc as plsc

_N = 10000
_E = 320000
_D = 128
_NUM_LAYERS = 3

_NC = 2
_NS = 16
_NW = _NC * _NS
_K = 64
_NBUF = 4
_LEAD = 2
_ISLOT = 8
_CHUNKS = 160
_EPAD = _NW * _K * _CHUNKS
_EW = _EPAD // _NW
_NPAD = 10240
_RPT = _NPAD // _NS
_ZC = 80
_NZ = _RPT // _ZC

_R = 1000
_G = _N // _R


def _spmm_body(m_hbm, eidx_hbm, out_hbm,
               ibuf, rows_v, bounce_v, acc_sh, gsems, ssems, isems):
    c = lax.axis_index("c")
    s = lax.axis_index("s")
    wid = c * _NS + s

    def zrow(n, _):
        for j in range(_D // 16):
            bounce_v[n, pl.ds(j * 16, 16)] = jnp.zeros((16,), jnp.float32)
        return 0
    lax.fori_loop(0, _ZC, zrow, 0)
    row0 = s * _RPT
    for z in range(_NZ):
        pltpu.sync_copy(bounce_v, acc_sh.at[pl.ds(row0 + z * _ZC, _ZC)])
    plsc.subcore_barrier()

    for k in range(_NBUF):
        pltpu.async_copy(eidx_hbm.at[wid, k], ibuf.at[k], isems.at[k])
    for b in range(_LEAD):
        pltpu.make_async_copy(
            eidx_hbm.at[wid, 0], ibuf.at[b], isems.at[b]).wait()
        pltpu.async_copy(m_hbm.at[ibuf.at[b, 0]], rows_v.at[b], gsems.at[b])

    def outer(t, _):
        for b in range(_NBUF):
            ci = t * _NBUF + b
            i8 = lax.rem(ci, _ISLOT)
            pltpu.make_async_copy(
                m_hbm.at[ibuf.at[i8, 0]], rows_v.at[b], gsems.at[b]).wait()

            eav_ref = ibuf.at[i8, 2]

            def grp(g, _):
                e0 = g * 16
                eav = lax.bitcast_convert_type(eav_ref[pl.ds(e0, 16)],
                                               jnp.float32)
                for i in range(16):
                    w = eav[i]
                    for j in range(_D // 16):
                        sl = pl.ds(j * 16, 16)
                        rows_v[b, e0 + i, sl] = rows_v[b, e0 + i, sl] * w
                return 0
            lax.fori_loop(0, _K // 16, grp, 0)
            pltpu.async_copy(rows_v.at[b], acc_sh.at[ibuf.at[i8, 1]],
                             ssems.at[b], add=True)

            nb = (b + _LEAD) % _NBUF
            nc2 = ci + _LEAD
            i82 = lax.rem(nc2, _ISLOT)

            @pl.when(nc2 < _CHUNKS)
            def _():
                @pl.when(nc2 >= _NBUF)
                def _():
                    pltpu.make_async_copy(
                        rows_v.at[nb], acc_sh.at[ibuf.at[0, 1]],
                        ssems.at[nb]).wait()
                pltpu.make_async_copy(
                    eidx_hbm.at[wid, 0], ibuf.at[i82], isems.at[i82]).wait()
                pltpu.async_copy(m_hbm.at[ibuf.at[i82, 0]], rows_v.at[nb],
                                 gsems.at[nb])

            nc4 = ci + _NBUF
            i84 = lax.rem(nc4, _ISLOT)

            @pl.when(nc4 < _CHUNKS)
            def _():
                pltpu.async_copy(eidx_hbm.at[wid, nc4], ibuf.at[i84],
                                 isems.at[i84])
        return 0
    lax.fori_loop(0, _CHUNKS // _NBUF, outer, 0)
    for b in range(_NBUF):
        pltpu.make_async_copy(
            rows_v.at[b], acc_sh.at[ibuf.at[0, 1]], ssems.at[b]).wait()
    plsc.subcore_barrier()

    for z in range(_NZ):
        r0 = row0 + z * _ZC
        pltpu.sync_copy(acc_sh.at[pl.ds(r0, _ZC)], bounce_v)
        pltpu.sync_copy(bounce_v, out_hbm.at[c, pl.ds(r0, _ZC)])


@jax.jit
def _spmm(m, eidx):
    mesh = plsc.VectorSubcoreMesh(core_axis_name="c", subcore_axis_name="s")
    return pl.kernel(
        _spmm_body,
        out_type=jax.ShapeDtypeStruct((_NC, _NPAD, _D), jnp.float32),
        mesh=mesh,
        scratch_types=[
            pltpu.VMEM((_ISLOT, 3, _K), jnp.int32),
            pltpu.VMEM((_NBUF, _K, _D), jnp.float32),
            pltpu.VMEM((_ZC, _D), jnp.float32),
            pltpu.VMEM_SHARED((_NPAD, _D), jnp.float32),
            pltpu.SemaphoreType.DMA((_NBUF,)),
            pltpu.SemaphoreType.DMA((_NBUF,)),
            pltpu.SemaphoreType.DMA((_ISLOT,)),
        ],
    )(m, eidx)


def _matmul_body(x_ref, w_ref, o_ref):
    o_ref[...] = jnp.dot(x_ref[...], w_ref[...],
                         preferred_element_type=jnp.float32)


@jax.jit
def _matmul(x, w):
    return pl.pallas_call(
        _matmul_body,
        grid=(_G,),
        in_specs=[
            pl.BlockSpec((_R, _D), lambda i: (i, 0)),
            pl.BlockSpec((_D, _D), lambda i: (0, 0)),
        ],
        out_specs=pl.BlockSpec((_R, _D), lambda i: (i, 0)),
        out_shape=jax.ShapeDtypeStruct((_N, _D), jnp.float32),
    )(x, w)


def _gru_body(s_ref, h_ref, wih_ref, whh_ref, bih_ref, bhh_ref, wn_ref,
              h_out, m_out):
    agg = s_ref[0] + s_ref[1]
    h = h_ref[...]
    gi = jnp.dot(agg, wih_ref[...], preferred_element_type=jnp.float32)
    gi = gi + bih_ref[...]
    gh = jnp.dot(h, whh_ref[...], preferred_element_type=jnp.float32)
    gh = gh + bhh_ref[...]
    r = jax.nn.sigmoid(gi[:, :_D] + gh[:, :_D])
    z = jax.nn.sigmoid(gi[:, _D:2 * _D] + gh[:, _D:2 * _D])
    n = jnp.tanh(gi[:, 2 * _D:] + r * gh[:, 2 * _D:])
    hn = (1.0 - z) * n + z * h
    h_out[...] = hn
    m_out[...] = jnp.dot(hn, wn_ref[...], preferred_element_type=jnp.float32)


@jax.jit
def _gru(s, h, wih_t, whh_t, bih, bhh, w_next):
    return pl.pallas_call(
        _gru_body,
        grid=(_G,),
        in_specs=[
            pl.BlockSpec((_NC, _R, _D), lambda i: (0, i, 0)),
            pl.BlockSpec((_R, _D), lambda i: (i, 0)),
            pl.BlockSpec((_D, 3 * _D), lambda i: (0, 0)),
            pl.BlockSpec((_D, 3 * _D), lambda i: (0, 0)),
            pl.BlockSpec((1, 3 * _D), lambda i: (0, 0)),
            pl.BlockSpec((1, 3 * _D), lambda i: (0, 0)),
            pl.BlockSpec((_D, _D), lambda i: (0, 0)),
        ],
        out_specs=[
            pl.BlockSpec((_R, _D), lambda i: (i, 0)),
            pl.BlockSpec((_R, _D), lambda i: (i, 0)),
        ],
        out_shape=[
            jax.ShapeDtypeStruct((_N, _D), jnp.float32),
            jax.ShapeDtypeStruct((_N, _D), jnp.float32),
        ],
    )(s, h, wih_t, whh_t, bih, bhh, w_next)


def _mlp_body(h_ref, w0_ref, b0_ref, w1_ref, b1_ref, w2_ref, b2_ref,
              wo_ref, bo_ref, emb_out, lsm_out):
    y = jnp.tanh(jnp.dot(h_ref[...], w0_ref[...],
                         preferred_element_type=jnp.float32) + b0_ref[...])
    y = jnp.tanh(jnp.dot(y, w1_ref[...],
                         preferred_element_type=jnp.float32) + b1_ref[...])
    y = jnp.tanh(jnp.dot(y, w2_ref[...],
                         preferred_element_type=jnp.float32) + b2_ref[...])
    logits = jnp.dot(y, wo_ref[...],
                     preferred_element_type=jnp.float32) + bo_ref[...]
    m = jnp.max(logits, axis=1, keepdims=True)
    lse = jnp.log(jnp.sum(jnp.exp(logits - m), axis=1, keepdims=True)) + m
    emb_out[...] = logits
    lsm_out[...] = logits - lse


@jax.jit
def _mlp(h, w0, b0, w1, b1, w2, b2, wo, bo):
    wspec = pl.BlockSpec((_D, _D), lambda i: (0, 0))
    bspec = pl.BlockSpec((1, _D), lambda i: (0, 0))
    ospec = pl.BlockSpec((_R, _D), lambda i: (i, 0))
    return pl.pallas_call(
        _mlp_body,
        grid=(_G,),
        in_specs=[ospec, wspec, bspec, wspec, bspec, wspec, bspec,
                  wspec, bspec],
        out_specs=[ospec, ospec],
        out_shape=[
            jax.ShapeDtypeStruct((_N, _D), jnp.float32),
            jax.ShapeDtypeStruct((_N, _D), jnp.float32),
        ],
    )(h, w0, b0, w1, b1, w2, b2, wo, bo)


def kernel(x, edge_index, edge_attr, W, W_ih, W_hh, b_ih, b_hh,
           mlp_w0, mlp_b0, mlp_w1, mlp_b1, mlp_w2, mlp_b2, out_w, out_b):
    f32 = jnp.float32
    src = edge_index[0].astype(jnp.int32)
    dst = edge_index[1].astype(jnp.int32)
    ea = edge_attr.astype(f32)
    pad = _EPAD - _E
    src_p = jnp.pad(src, (0, pad)).reshape(_NW, _CHUNKS, _K)
    dst_p = jnp.pad(dst, (0, pad)).reshape(_NW, _CHUNKS, _K)
    ea_p = jnp.pad(ea, (0, pad)).reshape(_NW, _CHUNKS, _K)
    eab = jax.lax.bitcast_convert_type(ea_p, jnp.int32)
    eidx = jnp.stack([src_p, dst_p, eab], axis=2)

    wih_t = W_ih.T.astype(f32)
    whh_t = W_hh.T.astype(f32)
    bih = b_ih.reshape(1, -1).astype(f32)
    bhh = b_hh.reshape(1, -1).astype(f32)

    h = x.astype(f32)
    m = _matmul(h, W[0].astype(f32))
    for i in range(_NUM_LAYERS):
        s = _spmm(m, eidx)[:, :_N]
        w_next = W[(i + 1) % _NUM_LAYERS].astype(f32)
        h, m = _gru(s, h, wih_t, whh_t, bih, bhh, w_next)

    mh = mlp_w0.shape[0]
    nc = out_w.shape[0]
    w0 = jnp.zeros((_D, _D), f32).at[:, :mh].set(mlp_w0.T)
    b0 = jnp.zeros((1, _D), f32).at[0, :mh].set(mlp_b0)
    w1 = jnp.zeros((_D, _D), f32).at[:mh, :mh].set(mlp_w1.T)
    b1 = jnp.zeros((1, _D), f32).at[0, :mh].set(mlp_b1)
    w2 = jnp.zeros((_D, _D), f32).at[:mh, :mh].set(mlp_w2.T)
    b2 = jnp.zeros((1, _D), f32).at[0, :mh].set(mlp_b2)
    wo = jnp.zeros((_D, _D), f32).at[:mh, :nc].set(out_w.T)
    bo = jnp.full((1, _D), -1e9, f32).at[0, :nc].set(out_b)
    emb, lsm = _mlp(h, w0, b0, w1, b1, w2, b2, wo, bo)
    return emb[:, :nc], lsm[:, :nc]
